# baseline (speedup 1.0000x reference)
import jax
import jax.numpy as jnp
from jax.experimental import pallas as pl
from jax.experimental.pallas import tpu as pltpu

_B = 8
_N = 512
_D = 512
_DA = _D + 8
_M = 16384
_K = 9
_T = 16
_BT = _B * _T

_TR = 1024
_TC = 2048
_NR = (_B * _N) // _TR
_NC = _M // _TC
_IPT = max(_TR // _N, 1)


def _build_body(y_ref, aug_ref, yn_ref):
    y32 = y_ref[...]
    ynorm = jnp.sum(y32 * y32, axis=0, keepdims=True)
    yn_ref[...] = ynorm
    hi = ynorm.astype(jnp.bfloat16)
    lo = (ynorm - hi.astype(jnp.float32)).astype(jnp.bfloat16)
    aug_ref[0:_D, :] = -2.0 * y32.astype(jnp.bfloat16)
    sub = jax.lax.broadcasted_iota(jnp.int32, (8, _TC), 0)
    tail = jnp.where(sub == 0, jnp.broadcast_to(hi, (8, _TC)),
                     jnp.where(sub == 1, jnp.broadcast_to(lo, (8, _TC)),
                               jnp.float32(0.0))).astype(jnp.bfloat16)
    aug_ref[_D:_DA, :] = tail


def _stage1_body(x_ref, y_ref, scores_ref, cand_ref, minval):
    i = pl.program_id(0)
    j = pl.program_id(1)

    @pl.when(j == 0)
    def _init():
        minval[...] = jnp.full_like(minval[...], jnp.inf)

    x = x_ref[...]
    s = jax.lax.dot_general(x, y_ref[...], (((1,), (0,)), ((), ())),
                            preferred_element_type=jnp.float32)
    tmin = jnp.min(s, axis=1, keepdims=True)
    minval[...] = jnp.minimum(minval[...], tmin)

    @pl.when(j == _NC - 1)
    def _finalize():
        x32 = x[:, 0:_D].astype(jnp.float32)
        xnorm = jnp.sum(x32 * x32, axis=1, keepdims=True)
        sc = jnp.sqrt(jnp.clip(xnorm + minval[...], 1e-12, None))
        scores_ref[...] = sc
        rowio = jax.lax.broadcasted_iota(jnp.int32, (_N, 1), 0)
        for bl in range(_IPT):
            b = i * _IPT + bl
            seg = sc[bl * _N:(bl + 1) * _N, :]
            for t in range(_T):
                m = jnp.argmax(seg[:, 0], axis=0).astype(jnp.int32)
                cand_ref[pl.ds(b * _T + t, 1), :] = (
                    jnp.full((1, 1), 0, jnp.int32) + b * _N + m)
                seg = jnp.where(rowio == m, -jnp.inf, seg)


def _gather_body(idx_ref, bank_ref, out_ref):
    out_ref[...] = bank_ref[...]


def _rescore_body(qc_ref, cr_ref, y_ref, yn_ref,
                  q_ref, nnidx_ref, maxsc_ref, dq_ref,
                  minv, mina, dsq):
    j = pl.program_id(0)

    @pl.when(j == 0)
    def _init():
        minv[...] = jnp.full_like(minv[...], jnp.inf)
        mina[...] = jnp.zeros_like(mina[...])

    qc = qc_ref[...]
    prod = jax.lax.dot_general(qc, y_ref[...], (((1,), (0,)), ((), ())),
                               preferred_element_type=jnp.float32)
    s = yn_ref[...] - 2.0 * prod
    dsq[:, pl.ds(j * _TC, _TC)] = s
    tmin = jnp.min(s, axis=1, keepdims=True)
    targ = jnp.argmin(s, axis=1).astype(jnp.int32)[:, None] + j * _TC
    better = tmin < minv[...]
    mina[...] = jnp.where(better, targ, mina[...])
    minv[...] = jnp.where(better, tmin, minv[...])

    @pl.when(j == _NC - 1)
    def _finalize():
        qnorm = jnp.sum(qc * qc, axis=1, keepdims=True)
        sc = jnp.sqrt(jnp.clip(qnorm + minv[...], 1e-12, None))
        rows = cr_ref[...]
        for b in range(_B):
            sl = slice(b * _T, (b + 1) * _T)
            seg = sc[sl, :]
            mx = jnp.max(seg)
            hit = seg == mx
            rseg = rows[sl, :]
            rsel = jnp.min(jnp.where(hit, rseg, _B * _N))
            selc = jnp.logical_and(hit, rseg == rsel)
            q_ref[pl.ds(b, 1), :] = jnp.sum(
                jnp.where(selc, qc[sl, :], 0.0), axis=0, keepdims=True)
            nnidx_ref[pl.ds(b, 1), :] = jnp.sum(
                jnp.where(selc, mina[sl, :], 0), axis=0, keepdims=True)
            maxsc_ref[pl.ds(b, 1), :] = jnp.full((1, 1), 0.0) + mx
            qn_sel = jnp.sum(jnp.where(selc, qnorm[sl, :], 0.0))
            dq_ref[pl.ds(b, 1), :] = qn_sel + jnp.sum(
                jnp.where(selc, dsq[sl, :], 0.0), axis=0, keepdims=True)


def _stage2b_body(nn_ref, dq_ref, maxsc_ref, y_ref, yn_ref, pred_ref, dn_sq):
    j = pl.program_id(0)
    nn = nn_ref[...]
    prod = jax.lax.dot_general(nn, y_ref[...], (((1,), (0,)), ((), ())),
                               preferred_element_type=jnp.float32)
    nnorm = jnp.sum(nn * nn, axis=1, keepdims=True)
    dn_sq[:, pl.ds(j * _TC, _TC)] = nnorm + (yn_ref[...] - 2.0 * prod)

    @pl.when(j == _NC - 1)
    def _finalize():
        dq = dq_ref[...]
        dn = dn_sq[...]
        colio = jax.lax.broadcasted_iota(jnp.int32, (_B, _M), 1)
        lane16 = jax.lax.broadcasted_iota(jnp.int32, (_B, 16), 1)
        dm = jnp.full((_B, 16), -jnp.inf, dtype=jnp.float32)
        for k in range(_K):
            midx = jnp.argmin(dn, axis=1).astype(jnp.int32)[:, None]
            onehot = colio == midx
            dq_k = jnp.sum(jnp.where(onehot, dq, 0.0), axis=1,
                           keepdims=True)
            dist_k = jnp.sqrt(jnp.clip(dq_k, 1e-12, None))
            dm = jnp.where(lane16 == k, jnp.broadcast_to(dist_k, (_B, 16)),
                           dm)
            dn = jnp.where(onehot, jnp.inf, dn)
        mx = jnp.max(dm, axis=1, keepdims=True)
        e = jnp.exp(dm - mx)
        w0 = e[:, 0:1] / jnp.sum(e, axis=1, keepdims=True)
        pred_ref[...] = (1.0 - w0) * maxsc_ref[...]


def _augment(v):
    bf16 = jnp.bfloat16
    n = v.shape[0]
    return jnp.concatenate(
        [v.astype(bf16), jnp.ones((n, 2), bf16), jnp.zeros((n, 6), bf16)],
        axis=1)


def _gather_rows(src, idx, n):
    r = src.shape[0]
    return pl.pallas_call(
        _gather_body,
        grid_spec=pltpu.PrefetchScalarGridSpec(
            num_scalar_prefetch=1,
            grid=(n,),
            in_specs=[pl.BlockSpec((1, 1, _D), lambda b, idx: (idx[b], 0, 0))],
            out_specs=pl.BlockSpec((1, 1, _D), lambda b, idx: (b, 0, 0)),
        ),
        out_shape=jax.ShapeDtypeStruct((n, 1, _D), jnp.float32),
    )(idx, src.reshape(r, 1, _D)).reshape(n, _D)


@jax.jit
def kernel(embedding, memory_bank):
    f32, i32 = jnp.float32, jnp.int32
    bank_t = memory_bank.T
    x_aug = _augment(embedding)

    y_aug, yn32 = pl.pallas_call(
        _build_body,
        grid=(_NC,),
        in_specs=[pl.BlockSpec((_D, _TC), lambda j: (0, j))],
        out_specs=[
            pl.BlockSpec((_DA, _TC), lambda j: (0, j)),
            pl.BlockSpec((1, _TC), lambda j: (0, j)),
        ],
        out_shape=[
            jax.ShapeDtypeStruct((_DA, _M), jnp.bfloat16),
            jax.ShapeDtypeStruct((1, _M), f32),
        ],
        compiler_params=pltpu.CompilerParams(
            dimension_semantics=("arbitrary",)),
    )(bank_t)

    scores, cand = pl.pallas_call(
        _stage1_body,
        grid=(_NR, _NC),
        in_specs=[
            pl.BlockSpec((_TR, _DA), lambda i, j: (i, 0)),
            pl.BlockSpec((_DA, _TC), lambda i, j: (0, j)),
        ],
        out_specs=[
            pl.BlockSpec((_TR, 1), lambda i, j: (i, 0)),
            pl.BlockSpec((_BT, 1), lambda i, j: (0, 0)),
        ],
        out_shape=[
            jax.ShapeDtypeStruct((_B * _N, 1), f32),
            jax.ShapeDtypeStruct((_BT, 1), i32),
        ],
        scratch_shapes=[pltpu.VMEM((_TR, 1), f32)],
        compiler_params=pltpu.CompilerParams(
            dimension_semantics=("arbitrary", "arbitrary")),
    )(x_aug, y_aug)

    q_cand = _gather_rows(embedding, cand.reshape(_BT), _BT)

    q8, nnidx, maxsc, dq = pl.pallas_call(
        _rescore_body,
        grid=(_NC,),
        in_specs=[
            pl.BlockSpec((_BT, _D), lambda j: (0, 0)),
            pl.BlockSpec((_BT, 1), lambda j: (0, 0)),
            pl.BlockSpec((_D, _TC), lambda j: (0, j)),
            pl.BlockSpec((1, _TC), lambda j: (0, j)),
        ],
        out_specs=[
            pl.BlockSpec((_B, _D), lambda j: (0, 0)),
            pl.BlockSpec((_B, 1), lambda j: (0, 0)),
            pl.BlockSpec((_B, 1), lambda j: (0, 0)),
            pl.BlockSpec((_B, _M), lambda j: (0, 0)),
        ],
        out_shape=[
            jax.ShapeDtypeStruct((_B, _D), f32),
            jax.ShapeDtypeStruct((_B, 1), i32),
            jax.ShapeDtypeStruct((_B, 1), f32),
            jax.ShapeDtypeStruct((_B, _M), f32),
        ],
        scratch_shapes=[
            pltpu.VMEM((_BT, 1), f32),
            pltpu.VMEM((_BT, 1), i32),
            pltpu.VMEM((_BT, _M), f32),
        ],
        compiler_params=pltpu.CompilerParams(
            dimension_semantics=("arbitrary",)),
    )(q_cand, cand, bank_t, yn32)

    nn8 = _gather_rows(memory_bank, nnidx.reshape(_B), _B)

    pred = pl.pallas_call(
        _stage2b_body,
        grid=(_NC,),
        in_specs=[
            pl.BlockSpec((_B, _D), lambda j: (0, 0)),
            pl.BlockSpec((_B, _M), lambda j: (0, 0)),
            pl.BlockSpec((_B, 1), lambda j: (0, 0)),
            pl.BlockSpec((_D, _TC), lambda j: (0, j)),
            pl.BlockSpec((1, _TC), lambda j: (0, j)),
        ],
        out_specs=pl.BlockSpec((_B, 1), lambda j: (0, 0)),
        out_shape=jax.ShapeDtypeStruct((_B, 1), f32),
        scratch_shapes=[pltpu.VMEM((_B, _M), f32)],
        compiler_params=pltpu.CompilerParams(
            dimension_semantics=("arbitrary",)),
    )(nn8, dq, maxsc, bank_t, yn32)

    return scores.reshape(_B, _N), pred.reshape(_B)


# speedup vs baseline: 1.1303x; 1.1303x over previous
import jax
import jax.numpy as jnp
from jax.experimental import pallas as pl
from jax.experimental.pallas import tpu as pltpu

_B = 8
_N = 512
_D = 512
_DA = _D + 8
_M = 16384
_K = 9
_T = 8
_BT = _B * _T

_TR = 1024
_TC = 2048
_NR = (_B * _N) // _TR
_NC = _M // _TC


def _build_body(y_ref, aug_ref, yn_ref):
    y32 = y_ref[...]
    ynorm = jnp.sum(y32 * y32, axis=0, keepdims=True)
    yn_ref[...] = ynorm
    hi = ynorm.astype(jnp.bfloat16)
    lo = (ynorm - hi.astype(jnp.float32)).astype(jnp.bfloat16)
    aug_ref[0:_D, :] = -2.0 * y32.astype(jnp.bfloat16)
    sub = jax.lax.broadcasted_iota(jnp.int32, (8, _TC), 0)
    tail = jnp.where(sub == 0, jnp.broadcast_to(hi, (8, _TC)),
                     jnp.where(sub == 1, jnp.broadcast_to(lo, (8, _TC)),
                               jnp.float32(0.0))).astype(jnp.bfloat16)
    aug_ref[_D:_DA, :] = tail


def _stage1_body(x_ref, y_ref, minout_ref):
    j = pl.program_id(1)
    s = jax.lax.dot_general(x_ref[...], y_ref[...], (((1,), (0,)), ((), ())),
                            preferred_element_type=jnp.float32)
    tmin = jnp.min(s, axis=1, keepdims=True)
    prev = jnp.where(jnp.full_like(tmin, j) == 0.0, jnp.inf, minout_ref[...])
    minout_ref[...] = jnp.minimum(prev, tmin)


def _finish_body(minv_ref, x_ref, scores_ref, cand_ref):
    x32 = x_ref[...]
    xnorm = jnp.sum(x32 * x32, axis=1, keepdims=True)
    sc = jnp.sqrt(jnp.clip(xnorm + minv_ref[...], 1e-12, None))
    scores_ref[...] = sc
    rowio = jax.lax.broadcasted_iota(jnp.int32, (_N, 1), 0)
    for b in range(_B):
        seg = sc[b * _N:(b + 1) * _N, :]
        for t in range(_T):
            m = jnp.argmax(seg[:, 0], axis=0).astype(jnp.int32)
            cand_ref[b * _T + t:b * _T + t + 1, :] = (
                jnp.full((1, 1), b * _N, jnp.int32) + m)
            seg = jnp.where(rowio == m, -jnp.inf, seg)


def _gather_body(idx_ref, bank_ref, out_ref):
    out_ref[...] = bank_ref[...]


def _rescore_body(qc_ref, y_ref, yn_ref, dsq_ref, minv_ref, mina_ref,
                  minv, mina):
    j = pl.program_id(0)

    @pl.when(j == 0)
    def _init():
        minv[...] = jnp.full_like(minv[...], jnp.inf)
        mina[...] = jnp.zeros_like(mina[...])

    prod = jax.lax.dot_general(qc_ref[...], y_ref[...],
                               (((1,), (0,)), ((), ())),
                               preferred_element_type=jnp.float32)
    s = yn_ref[...] - 2.0 * prod
    dsq_ref[:, pl.ds(j * _TC, _TC)] = s
    tmin = jnp.min(s, axis=1, keepdims=True)
    targ = jnp.argmin(s, axis=1).astype(jnp.int32)[:, None] + j * _TC
    better = tmin < minv[...]
    mina[...] = jnp.where(better, targ, mina[...])
    minv[...] = jnp.where(better, tmin, minv[...])
    minv_ref[...] = minv[...]
    mina_ref[...] = mina[...]


def _select_body(qc_ref, cr_ref, minv_ref, mina_ref, dsq_ref,
                 q_ref, nnidx_ref, maxsc_ref, dq_ref):
    qc = qc_ref[...]
    qnorm = jnp.sum(qc * qc, axis=1, keepdims=True)
    sc = jnp.sqrt(jnp.clip(qnorm + minv_ref[...], 1e-12, None))
    rows = cr_ref[...]
    for b in range(_B):
        sl = slice(b * _T, (b + 1) * _T)
        seg = sc[sl, :]
        mx = jnp.max(seg)
        hit = seg == mx
        rseg = rows[sl, :]
        rsel = jnp.min(jnp.where(hit, rseg, _B * _N))
        selc = jnp.logical_and(hit, rseg == rsel)
        q_ref[b:b + 1, :] = jnp.sum(
            jnp.where(selc, qc[sl, :], 0.0), axis=0, keepdims=True)
        nnidx_ref[b:b + 1, :] = jnp.sum(
            jnp.where(selc, mina_ref[sl, :], 0), axis=0, keepdims=True)
        maxsc_ref[b:b + 1, :] = jnp.full((1, 1), 0.0) + mx
        qn_sel = jnp.sum(jnp.where(selc, qnorm[sl, :], 0.0))
        dq_ref[b:b + 1, :] = qn_sel + jnp.sum(
            jnp.where(selc, dsq_ref[sl, :], 0.0), axis=0, keepdims=True)


def _dn_body(nn_ref, y_ref, yn_ref, dn_ref):
    j = pl.program_id(0)
    nn = nn_ref[...]
    prod = jax.lax.dot_general(nn, y_ref[...], (((1,), (0,)), ((), ())),
                               preferred_element_type=jnp.float32)
    nnorm = jnp.sum(nn * nn, axis=1, keepdims=True)
    dn_ref[:, pl.ds(j * _TC, _TC)] = nnorm + (yn_ref[...] - 2.0 * prod)


def _pred_body(dn_ref, dq_ref, maxsc_ref, pred_ref):
    dq = dq_ref[...]
    dn = dn_ref[...]
    colio = jax.lax.broadcasted_iota(jnp.int32, (_B, _M), 1)
    lane16 = jax.lax.broadcasted_iota(jnp.int32, (_B, 16), 1)
    dm = jnp.full((_B, 16), -jnp.inf, dtype=jnp.float32)
    for k in range(_K):
        midx = jnp.argmin(dn, axis=1).astype(jnp.int32)[:, None]
        onehot = colio == midx
        dq_k = jnp.sum(jnp.where(onehot, dq, 0.0), axis=1, keepdims=True)
        dist_k = jnp.sqrt(jnp.clip(dq_k, 1e-12, None))
        dm = jnp.where(lane16 == k, jnp.broadcast_to(dist_k, (_B, 16)), dm)
        dn = jnp.where(onehot, jnp.inf, dn)
    mx = jnp.max(dm, axis=1, keepdims=True)
    e = jnp.exp(dm - mx)
    w0 = e[:, 0:1] / jnp.sum(e, axis=1, keepdims=True)
    pred_ref[...] = (1.0 - w0) * maxsc_ref[...]


def _augment(v):
    bf16 = jnp.bfloat16
    n = v.shape[0]
    return jnp.concatenate(
        [v.astype(bf16), jnp.ones((n, 2), bf16), jnp.zeros((n, 6), bf16)],
        axis=1)


def _gather_rows(src, idx, n):
    r = src.shape[0]
    return pl.pallas_call(
        _gather_body,
        grid_spec=pltpu.PrefetchScalarGridSpec(
            num_scalar_prefetch=1,
            grid=(n,),
            in_specs=[pl.BlockSpec((1, 1, _D), lambda b, idx: (idx[b], 0, 0))],
            out_specs=pl.BlockSpec((1, 1, _D), lambda b, idx: (b, 0, 0)),
        ),
        out_shape=jax.ShapeDtypeStruct((n, 1, _D), jnp.float32),
    )(idx, src.reshape(r, 1, _D)).reshape(n, _D)


@jax.jit
def kernel(embedding, memory_bank):
    f32, i32 = jnp.float32, jnp.int32
    bank_t = memory_bank.T
    x_aug = _augment(embedding)

    y_aug, yn32 = pl.pallas_call(
        _build_body,
        grid=(_NC,),
        in_specs=[pl.BlockSpec((_D, _TC), lambda j: (0, j))],
        out_specs=[
            pl.BlockSpec((_DA, _TC), lambda j: (0, j)),
            pl.BlockSpec((1, _TC), lambda j: (0, j)),
        ],
        out_shape=[
            jax.ShapeDtypeStruct((_DA, _M), jnp.bfloat16),
            jax.ShapeDtypeStruct((1, _M), f32),
        ],
        compiler_params=pltpu.CompilerParams(
            dimension_semantics=("arbitrary",)),
    )(bank_t)

    minvals = pl.pallas_call(
        _stage1_body,
        grid=(_NR, _NC),
        in_specs=[
            pl.BlockSpec((_TR, _DA), lambda i, j: (i, 0)),
            pl.BlockSpec((_DA, _TC), lambda i, j: (0, j)),
        ],
        out_specs=pl.BlockSpec((_TR, 1), lambda i, j: (i, 0)),
        out_shape=jax.ShapeDtypeStruct((_B * _N, 1), f32),
        compiler_params=pltpu.CompilerParams(
            dimension_semantics=("arbitrary", "arbitrary")),
    )(x_aug, y_aug)

    scores, cand = pl.pallas_call(
        _finish_body,
        grid=(1,),
        in_specs=[
            pl.BlockSpec((_B * _N, 1), lambda z: (0, 0)),
            pl.BlockSpec((_B * _N, _D), lambda z: (0, 0)),
        ],
        out_specs=[
            pl.BlockSpec((_B * _N, 1), lambda z: (0, 0)),
            pl.BlockSpec((_BT, 1), lambda z: (0, 0)),
        ],
        out_shape=[
            jax.ShapeDtypeStruct((_B * _N, 1), f32),
            jax.ShapeDtypeStruct((_BT, 1), i32),
        ],
    )(minvals, embedding)

    q_cand = _gather_rows(embedding, cand.reshape(_BT), _BT)

    dsq, minv, mina = pl.pallas_call(
        _rescore_body,
        grid=(_NC,),
        in_specs=[
            pl.BlockSpec((_BT, _D), lambda j: (0, 0)),
            pl.BlockSpec((_D, _TC), lambda j: (0, j)),
            pl.BlockSpec((1, _TC), lambda j: (0, j)),
        ],
        out_specs=[
            pl.BlockSpec((_BT, _M), lambda j: (0, 0)),
            pl.BlockSpec((_BT, 1), lambda j: (0, 0)),
            pl.BlockSpec((_BT, 1), lambda j: (0, 0)),
        ],
        out_shape=[
            jax.ShapeDtypeStruct((_BT, _M), f32),
            jax.ShapeDtypeStruct((_BT, 1), f32),
            jax.ShapeDtypeStruct((_BT, 1), i32),
        ],
        scratch_shapes=[
            pltpu.VMEM((_BT, 1), f32),
            pltpu.VMEM((_BT, 1), i32),
        ],
        compiler_params=pltpu.CompilerParams(
            dimension_semantics=("arbitrary",)),
    )(q_cand, bank_t, yn32)

    q8, nnidx, maxsc, dq = pl.pallas_call(
        _select_body,
        grid=(1,),
        in_specs=[
            pl.BlockSpec((_BT, _D), lambda z: (0, 0)),
            pl.BlockSpec((_BT, 1), lambda z: (0, 0)),
            pl.BlockSpec((_BT, 1), lambda z: (0, 0)),
            pl.BlockSpec((_BT, 1), lambda z: (0, 0)),
            pl.BlockSpec((_BT, _M), lambda z: (0, 0)),
        ],
        out_specs=[
            pl.BlockSpec((_B, _D), lambda z: (0, 0)),
            pl.BlockSpec((_B, 1), lambda z: (0, 0)),
            pl.BlockSpec((_B, 1), lambda z: (0, 0)),
            pl.BlockSpec((_B, _M), lambda z: (0, 0)),
        ],
        out_shape=[
            jax.ShapeDtypeStruct((_B, _D), f32),
            jax.ShapeDtypeStruct((_B, 1), i32),
            jax.ShapeDtypeStruct((_B, 1), f32),
            jax.ShapeDtypeStruct((_B, _M), f32),
        ],
    )(q_cand, cand, minv, mina, dsq)

    nn8 = _gather_rows(memory_bank, nnidx.reshape(_B), _B)

    dn = pl.pallas_call(
        _dn_body,
        grid=(_NC,),
        in_specs=[
            pl.BlockSpec((_B, _D), lambda j: (0, 0)),
            pl.BlockSpec((_D, _TC), lambda j: (0, j)),
            pl.BlockSpec((1, _TC), lambda j: (0, j)),
        ],
        out_specs=pl.BlockSpec((_B, _M), lambda j: (0, 0)),
        out_shape=jax.ShapeDtypeStruct((_B, _M), f32),
        compiler_params=pltpu.CompilerParams(
            dimension_semantics=("arbitrary",)),
    )(nn8, bank_t, yn32)

    pred = pl.pallas_call(
        _pred_body,
        grid=(1,),
        in_specs=[
            pl.BlockSpec((_B, _M), lambda z: (0, 0)),
            pl.BlockSpec((_B, _M), lambda z: (0, 0)),
            pl.BlockSpec((_B, 1), lambda z: (0, 0)),
        ],
        out_specs=pl.BlockSpec((_B, 1), lambda z: (0, 0)),
        out_shape=jax.ShapeDtypeStruct((_B, 1), f32),
    )(dn, dq, maxsc)

    return scores.reshape(_B, _N), pred.reshape(_B)


# speedup vs baseline: 1.3152x; 1.1636x over previous
import jax
import jax.numpy as jnp
from jax.experimental import pallas as pl
from jax.experimental.pallas import tpu as pltpu

_B = 8
_N = 512
_D = 512
_DA = _D + 8
_M = 16384
_K = 9
_T = 8
_BT = _B * _T

_TR = 1024
_TC = 2048
_NR = (_B * _N) // _TR
_NC = _M // _TC
_IPT = _TR // _N


def _build_body(y_ref, aug_ref, yn_ref):
    y32 = y_ref[...]
    ynorm = jnp.sum(y32 * y32, axis=0, keepdims=True)
    yn_ref[...] = ynorm
    hi = ynorm.astype(jnp.bfloat16)
    lo = (ynorm - hi.astype(jnp.float32)).astype(jnp.bfloat16)
    aug_ref[0:_D, :] = -2.0 * y32.astype(jnp.bfloat16)
    sub = jax.lax.broadcasted_iota(jnp.int32, (8, _TC), 0)
    tail = jnp.where(sub == 0, jnp.broadcast_to(hi, (8, _TC)),
                     jnp.where(sub == 1, jnp.broadcast_to(lo, (8, _TC)),
                               jnp.float32(0.0))).astype(jnp.bfloat16)
    aug_ref[_D:_DA, :] = tail


def _stage1_body(xa_ref, x32_ref, y_ref, scores_ref, cand_ref, qc_ref,
                 minval):
    i = pl.program_id(0)
    j = pl.program_id(1)

    @pl.when(j == 0)
    def _init():
        minval[...] = jnp.full_like(minval[...], jnp.inf)

    s = jax.lax.dot_general(xa_ref[...], y_ref[...], (((1,), (0,)), ((), ())),
                            preferred_element_type=jnp.float32)
    tmin = jnp.min(s, axis=1, keepdims=True)
    minval[...] = jnp.minimum(minval[...], tmin)

    @pl.when(j == _NC - 1)
    def _finalize():
        x32 = x32_ref[...]
        xnorm = jnp.sum(x32 * x32, axis=1, keepdims=True)
        sc = jnp.sqrt(jnp.clip(xnorm + minval[...], 1e-12, None))
        scores_ref[...] = sc
        rowio = jax.lax.broadcasted_iota(jnp.int32, (_N, 1), 0)
        for bl in range(_IPT):
            b = i * _IPT + bl
            seg = sc[bl * _N:(bl + 1) * _N, :]
            xseg = x32[bl * _N:(bl + 1) * _N, :]
            for t in range(_T):
                m = jnp.argmax(seg[:, 0], axis=0).astype(jnp.int32)
                sel = rowio == m
                cand_ref[pl.ds(b * _T + t, 1), :] = (
                    jnp.full((1, 1), 0, jnp.int32) + b * _N + m)
                qc_ref[pl.ds(b * _T + t, 1), :] = jnp.sum(
                    jnp.where(sel, xseg, 0.0), axis=0, keepdims=True)
                seg = jnp.where(sel, -jnp.inf, seg)


def _rescore_body(qc_ref, cr_ref, y_ref, yn_ref,
                  nnidx_ref, maxsc_ref, dq_ref,
                  minv, mina, dsq):
    j = pl.program_id(0)

    @pl.when(j == 0)
    def _init():
        minv[...] = jnp.full_like(minv[...], jnp.inf)
        mina[...] = jnp.zeros_like(mina[...])

    qc = qc_ref[...]
    prod = jax.lax.dot_general(qc, y_ref[...], (((1,), (0,)), ((), ())),
                               preferred_element_type=jnp.float32)
    s = yn_ref[...] - 2.0 * prod
    dsq[:, pl.ds(j * _TC, _TC)] = s
    tmin = jnp.min(s, axis=1, keepdims=True)
    targ = jnp.argmin(s, axis=1).astype(jnp.int32)[:, None] + j * _TC
    better = tmin < minv[...]
    mina[...] = jnp.where(better, targ, mina[...])
    minv[...] = jnp.where(better, tmin, minv[...])

    @pl.when(j == _NC - 1)
    def _finalize():
        qnorm = jnp.sum(qc * qc, axis=1, keepdims=True)
        sc = jnp.sqrt(jnp.clip(qnorm + minv[...], 1e-12, None))
        rows = cr_ref[...]
        for b in range(_B):
            sl = slice(b * _T, (b + 1) * _T)
            seg = sc[sl, :]
            mx = jnp.max(seg)
            hit = seg == mx
            rseg = rows[sl, :]
            rsel = jnp.min(jnp.where(hit, rseg, _B * _N))
            selc = jnp.logical_and(hit, rseg == rsel)
            nnidx_ref[b:b + 1, :] = jnp.sum(
                jnp.where(selc, mina[sl, :], 0), axis=0, keepdims=True)
            maxsc_ref[b:b + 1, :] = jnp.full((1, 1), 0.0) + mx
            qn_sel = jnp.sum(jnp.where(selc, qnorm[sl, :], 0.0))
            dq_ref[b:b + 1, :] = qn_sel + jnp.sum(
                jnp.where(selc, dsq[sl, :], 0.0), axis=0, keepdims=True)


def _gather_body(idx_ref, bank_ref, out_ref):
    out_ref[...] = bank_ref[...]


def _support_body(nn_ref, dq_ref, maxsc_ref, y_ref, yn_ref, pred_ref, dn_sq):
    j = pl.program_id(0)
    nn = nn_ref[...]
    prod = jax.lax.dot_general(nn, y_ref[...], (((1,), (0,)), ((), ())),
                               preferred_element_type=jnp.float32)
    nnorm = jnp.sum(nn * nn, axis=1, keepdims=True)
    dn_sq[:, pl.ds(j * _TC, _TC)] = nnorm + (yn_ref[...] - 2.0 * prod)

    @pl.when(j == _NC - 1)
    def _finalize():
        dq = dq_ref[...]
        dn = dn_sq[...]
        colio = jax.lax.broadcasted_iota(jnp.int32, (_B, _M), 1)
        lane16 = jax.lax.broadcasted_iota(jnp.int32, (_B, 16), 1)
        dm = jnp.full((_B, 16), -jnp.inf, dtype=jnp.float32)
        for k in range(_K):
            midx = jnp.argmin(dn, axis=1).astype(jnp.int32)[:, None]
            onehot = colio == midx
            dq_k = jnp.sum(jnp.where(onehot, dq, 0.0), axis=1,
                           keepdims=True)
            dist_k = jnp.sqrt(jnp.clip(dq_k, 1e-12, None))
            dm = jnp.where(lane16 == k, jnp.broadcast_to(dist_k, (_B, 16)),
                           dm)
            dn = jnp.where(onehot, jnp.inf, dn)
        mx = jnp.max(dm, axis=1, keepdims=True)
        e = jnp.exp(dm - mx)
        w0 = e[:, 0:1] / jnp.sum(e, axis=1, keepdims=True)
        pred_ref[...] = (1.0 - w0) * maxsc_ref[...]


def _augment(v):
    bf16 = jnp.bfloat16
    n = v.shape[0]
    return jnp.concatenate(
        [v.astype(bf16), jnp.ones((n, 2), bf16), jnp.zeros((n, 6), bf16)],
        axis=1)


@jax.jit
def kernel(embedding, memory_bank):
    f32, i32 = jnp.float32, jnp.int32
    bank_t = memory_bank.T
    x_aug = _augment(embedding)

    y_aug, yn32 = pl.pallas_call(
        _build_body,
        grid=(_NC,),
        in_specs=[pl.BlockSpec((_D, _TC), lambda j: (0, j))],
        out_specs=[
            pl.BlockSpec((_DA, _TC), lambda j: (0, j)),
            pl.BlockSpec((1, _TC), lambda j: (0, j)),
        ],
        out_shape=[
            jax.ShapeDtypeStruct((_DA, _M), jnp.bfloat16),
            jax.ShapeDtypeStruct((1, _M), f32),
        ],
        compiler_params=pltpu.CompilerParams(
            dimension_semantics=("arbitrary",)),
    )(bank_t)

    scores, cand, q_cand = pl.pallas_call(
        _stage1_body,
        grid=(_NR, _NC),
        in_specs=[
            pl.BlockSpec((_TR, _DA), lambda i, j: (i, 0)),
            pl.BlockSpec((_TR, _D), lambda i, j: (i, 0)),
            pl.BlockSpec((_DA, _TC), lambda i, j: (0, j)),
        ],
        out_specs=[
            pl.BlockSpec((_TR, 1), lambda i, j: (i, 0)),
            pl.BlockSpec((_BT, 1), lambda i, j: (0, 0)),
            pl.BlockSpec((_BT, _D), lambda i, j: (0, 0)),
        ],
        out_shape=[
            jax.ShapeDtypeStruct((_B * _N, 1), f32),
            jax.ShapeDtypeStruct((_BT, 1), i32),
            jax.ShapeDtypeStruct((_BT, _D), f32),
        ],
        scratch_shapes=[pltpu.VMEM((_TR, 1), f32)],
        compiler_params=pltpu.CompilerParams(
            dimension_semantics=("arbitrary", "arbitrary")),
    )(x_aug, embedding, y_aug)

    nnidx, maxsc, dq = pl.pallas_call(
        _rescore_body,
        grid=(_NC,),
        in_specs=[
            pl.BlockSpec((_BT, _D), lambda j: (0, 0)),
            pl.BlockSpec((_BT, 1), lambda j: (0, 0)),
            pl.BlockSpec((_D, _TC), lambda j: (0, j)),
            pl.BlockSpec((1, _TC), lambda j: (0, j)),
        ],
        out_specs=[
            pl.BlockSpec((_B, 1), lambda j: (0, 0)),
            pl.BlockSpec((_B, 1), lambda j: (0, 0)),
            pl.BlockSpec((_B, _M), lambda j: (0, 0)),
        ],
        out_shape=[
            jax.ShapeDtypeStruct((_B, 1), i32),
            jax.ShapeDtypeStruct((_B, 1), f32),
            jax.ShapeDtypeStruct((_B, _M), f32),
        ],
        scratch_shapes=[
            pltpu.VMEM((_BT, 1), f32),
            pltpu.VMEM((_BT, 1), i32),
            pltpu.VMEM((_BT, _M), f32),
        ],
        compiler_params=pltpu.CompilerParams(
            dimension_semantics=("arbitrary",)),
    )(q_cand, cand, bank_t, yn32)

    nn8 = pl.pallas_call(
        _gather_body,
        grid_spec=pltpu.PrefetchScalarGridSpec(
            num_scalar_prefetch=1,
            grid=(_B,),
            in_specs=[pl.BlockSpec((1, 1, _D), lambda b, idx: (idx[b], 0, 0))],
            out_specs=pl.BlockSpec((1, 1, _D), lambda b, idx: (b, 0, 0)),
        ),
        out_shape=jax.ShapeDtypeStruct((_B, 1, _D), f32),
    )(nnidx.reshape(_B), memory_bank.reshape(_M, 1, _D)).reshape(_B, _D)

    pred = pl.pallas_call(
        _support_body,
        grid=(_NC,),
        in_specs=[
            pl.BlockSpec((_B, _D), lambda j: (0, 0)),
            pl.BlockSpec((_B, _M), lambda j: (0, 0)),
            pl.BlockSpec((_B, 1), lambda j: (0, 0)),
            pl.BlockSpec((_D, _TC), lambda j: (0, j)),
            pl.BlockSpec((1, _TC), lambda j: (0, j)),
        ],
        out_specs=pl.BlockSpec((_B, 1), lambda j: (0, 0)),
        out_shape=jax.ShapeDtypeStruct((_B, 1), f32),
        scratch_shapes=[pltpu.VMEM((_B, _M), f32)],
        compiler_params=pltpu.CompilerParams(
            dimension_semantics=("arbitrary",)),
    )(nn8, dq, maxsc, bank_t, yn32)

    return scores.reshape(_B, _N), pred.reshape(_B)
